# baseline (speedup 1.0000x reference)
import functools

import jax
import jax.numpy as jnp
from jax import lax
from jax.experimental import pallas as pl
from jax.experimental.pallas import tpu as pltpu
from jax.experimental.pallas import tpu_sc as plsc

VOCAB = 1000000
EMBED = 64
B_ROWS = 4096
B_COLS = 50
TOTAL = B_ROWS * B_COLS
CHUNK = 128
N_CHUNK_ROWS = TOTAL // CHUNK

_info = plsc.get_sparse_core_info()
NC, NS = _info.num_cores, _info.num_subcores
NW = NC * NS
ROWS_PER_W = N_CHUNK_ROWS // NW


def _make_kernel():
    mesh = plsc.VectorSubcoreMesh(core_axis_name="c", subcore_axis_name="s")

    @functools.partial(
        pl.kernel,
        mesh=mesh,
        compiler_params=pltpu.CompilerParams(use_tc_tiling_on_sc=False),
        out_type=jax.ShapeDtypeStruct((TOTAL, EMBED), jnp.float32),
        scratch_types=[
            pltpu.VMEM((ROWS_PER_W, CHUNK), jnp.int32),
            pltpu.VMEM((CHUNK, EMBED), jnp.float32),
            pltpu.SemaphoreType.DMA,
        ],
    )
    def k(idx_hbm, table_hbm, out_hbm, idx_v, rows_v, sem):
        wid = lax.axis_index("s") * NC + lax.axis_index("c")
        out_base = wid * ROWS_PER_W * CHUNK

        pltpu.sync_copy(idx_hbm.at[wid], idx_v)

        def body(j, _):
            pltpu.async_copy(table_hbm.at[idx_v.at[j]], rows_v, sem).wait()
            off = pl.multiple_of(out_base + j * CHUNK, CHUNK)
            pltpu.sync_copy(rows_v, out_hbm.at[pl.ds(off, CHUNK)])
            return 0

        lax.fori_loop(0, ROWS_PER_W, body, 0)

    return k


_kernel_call = _make_kernel()


def kernel(inputs, embeddings):
    idx = jnp.reshape(inputs.astype(jnp.int32), (NW, ROWS_PER_W, CHUNK))
    out = _kernel_call(idx, embeddings)
    return jnp.reshape(out, (B_ROWS, B_COLS, EMBED))


# speedup vs baseline: 1.0444x; 1.0444x over previous
import functools

import jax
import jax.numpy as jnp
from jax import lax
from jax.experimental import pallas as pl
from jax.experimental.pallas import tpu as pltpu
from jax.experimental.pallas import tpu_sc as plsc

VOCAB = 1000000
EMBED = 64
B_ROWS = 4096
B_COLS = 50
TOTAL = B_ROWS * B_COLS
CHUNK = 128
N_CHUNK_ROWS = TOTAL // CHUNK

_info = plsc.get_sparse_core_info()
NC, NS = _info.num_cores, _info.num_subcores
NW = NC * NS
ROWS_PER_W = N_CHUNK_ROWS // NW
NBUF = 10


def _make_kernel():
    mesh = plsc.VectorSubcoreMesh(core_axis_name="c", subcore_axis_name="s")

    @functools.partial(
        pl.kernel,
        mesh=mesh,
        compiler_params=pltpu.CompilerParams(use_tc_tiling_on_sc=False),
        out_type=jax.ShapeDtypeStruct((TOTAL, EMBED), jnp.float32),
        scratch_types=[
            pltpu.VMEM((ROWS_PER_W, CHUNK), jnp.int32),
            pltpu.VMEM((NBUF, CHUNK, EMBED), jnp.float32),
            [pltpu.SemaphoreType.DMA] * NBUF,
        ],
    )
    def k(idx_hbm, table_hbm, out_hbm, idx_v, rows_v, sems):
        wid = lax.axis_index("s") * NC + lax.axis_index("c")
        out_base = wid * ROWS_PER_W * CHUNK

        pltpu.sync_copy(idx_hbm.at[wid], idx_v)

        for b in range(NBUF):
            pltpu.async_copy(table_hbm.at[idx_v.at[b]], rows_v.at[b], sems[b])

        @pl.loop(0, ROWS_PER_W, step=NBUF)
        def _ring(g0):
            for b in range(NBUF):
                g = g0 + b
                pltpu.make_async_copy(table_hbm.at[idx_v.at[g]], rows_v.at[b],
                                      sems[b]).wait()
                off = pl.multiple_of(out_base + g * CHUNK, CHUNK)
                pltpu.sync_copy(rows_v.at[b], out_hbm.at[pl.ds(off, CHUNK)])
                nxt = g + NBUF

                @pl.when(nxt < ROWS_PER_W)
                def _():
                    pltpu.async_copy(table_hbm.at[idx_v.at[nxt]], rows_v.at[b],
                                     sems[b])

    return k


_kernel_call = _make_kernel()


def kernel(inputs, embeddings):
    idx = jnp.reshape(inputs.astype(jnp.int32), (NW, ROWS_PER_W, CHUNK))
    out = _kernel_call(idx, embeddings)
    return jnp.reshape(out, (B_ROWS, B_COLS, EMBED))
